# baseline (speedup 1.0000x reference)
import jax
import jax.numpy as jnp
from jax.experimental import pallas as pl
from jax.experimental.pallas import tpu as pltpu

_B = 4
_S = 2048
_D = 768
_ROWS = _B * _S
_RB = 1024
_BLKS_PER_BATCH = _S // _RB
_NBLK = _ROWS // _RB


def _onehot(b):
    return jnp.where(
        jax.lax.broadcasted_iota(jnp.int32, (_B, 1), 0) == b, 1.0, 0.0)


def _body(xa_ref, xb_ref, qa_ref, qb_ref, o_ref, accx, accq):
    i = pl.program_id(0)
    n = pl.num_programs(0)

    @pl.when(i == 0)
    def _init():
        accx[...] = jnp.zeros_like(accx)
        accq[...] = jnp.zeros_like(accq)

    lo = _onehot(i // _BLKS_PER_BATCH)
    hi = _onehot((i + _NBLK // 2) // _BLKS_PER_BATCH)
    accx[...] += (lo * jnp.sum(xa_ref[...], axis=1)
                  + hi * jnp.sum(xb_ref[...], axis=1))
    accq[...] += (lo * jnp.sum(qa_ref[...], axis=1)
                  + hi * jnp.sum(qb_ref[...], axis=1))

    @pl.when(i == n - 1)
    def _finish():
        xp = accx[...] * (1.0 / _S)
        qp = accq[...] * (1.0 / _S)
        attn = jax.lax.dot_general(
            qp, xp, (((1,), (1,)), ((), ())),
            preferred_element_type=jnp.float32)
        attn = jax.nn.softmax(attn, axis=-1)
        ctx = jnp.dot(attn, xp, preferred_element_type=jnp.float32)
        o_ref[...] = ctx[:, None, :]


def kernel(x, query, keys, values, energy_score):
    del keys, values, energy_score
    return pl.pallas_call(
        _body,
        grid=(_NBLK // 2,),
        in_specs=[
            pl.BlockSpec((1, _RB, _D), lambda i: (i, 0, 0)),
            pl.BlockSpec((1, _RB, _D), lambda i: (i + _NBLK // 2, 0, 0)),
            pl.BlockSpec((1, _RB, _D), lambda i: (i, 0, 0)),
            pl.BlockSpec((1, _RB, _D), lambda i: (i + _NBLK // 2, 0, 0)),
        ],
        out_specs=pl.BlockSpec((_B, 1, _D), lambda i: (0, 0, 0)),
        out_shape=jax.ShapeDtypeStruct((_B, 1, _D), jnp.float32),
        scratch_shapes=[
            pltpu.VMEM((_B, _D), jnp.float32),
            pltpu.VMEM((_B, _D), jnp.float32),
        ],
    )(x.reshape(_NBLK, _RB, _D), x.reshape(_NBLK, _RB, _D),
      query.reshape(_NBLK, _RB, _D), query.reshape(_NBLK, _RB, _D))


# speedup vs baseline: 1.0244x; 1.0244x over previous
import jax
import jax.numpy as jnp
from jax.experimental import pallas as pl
from jax.experimental.pallas import tpu as pltpu

_B = 4
_S = 2048
_D = 768
_ROWS = _B * _S
_RB = 512
_BLKS_PER_BATCH = _S // _RB
_NBLK = _ROWS // _RB


def _onehot(b):
    return jnp.where(
        jax.lax.broadcasted_iota(jnp.int32, (_B, 1), 0) == b, 1.0, 0.0)


def _body(xa_ref, xb_ref, qa_ref, qb_ref, o_ref, accx, accq):
    i = pl.program_id(0)
    n = pl.num_programs(0)

    @pl.when(i == 0)
    def _init():
        accx[...] = jnp.zeros_like(accx)
        accq[...] = jnp.zeros_like(accq)

    lo = _onehot(i // _BLKS_PER_BATCH)
    hi = _onehot((i + _NBLK // 2) // _BLKS_PER_BATCH)
    accx[...] += (lo * jnp.sum(xa_ref[...], axis=1)
                  + hi * jnp.sum(xb_ref[...], axis=1))
    accq[...] += (lo * jnp.sum(qa_ref[...], axis=1)
                  + hi * jnp.sum(qb_ref[...], axis=1))

    @pl.when(i == n - 1)
    def _finish():
        xp = accx[...] * (1.0 / _S)
        qp = accq[...] * (1.0 / _S)
        attn = jax.lax.dot_general(
            qp, xp, (((1,), (1,)), ((), ())),
            preferred_element_type=jnp.float32)
        attn = jax.nn.softmax(attn, axis=-1)
        ctx = jnp.dot(attn, xp, preferred_element_type=jnp.float32)
        o_ref[...] = ctx[:, None, :]


def kernel(x, query, keys, values, energy_score):
    del keys, values, energy_score
    return pl.pallas_call(
        _body,
        grid=(_NBLK // 2,),
        in_specs=[
            pl.BlockSpec((1, _RB, _D), lambda i: (i, 0, 0)),
            pl.BlockSpec((1, _RB, _D), lambda i: (i + _NBLK // 2, 0, 0)),
            pl.BlockSpec((1, _RB, _D), lambda i: (i, 0, 0)),
            pl.BlockSpec((1, _RB, _D), lambda i: (i + _NBLK // 2, 0, 0)),
        ],
        out_specs=pl.BlockSpec((_B, 1, _D), lambda i: (0, 0, 0)),
        out_shape=jax.ShapeDtypeStruct((_B, 1, _D), jnp.float32),
        scratch_shapes=[
            pltpu.VMEM((_B, _D), jnp.float32),
            pltpu.VMEM((_B, _D), jnp.float32),
        ],
    )(x.reshape(_NBLK, _RB, _D), x.reshape(_NBLK, _RB, _D),
      query.reshape(_NBLK, _RB, _D), query.reshape(_NBLK, _RB, _D))


# speedup vs baseline: 1.0300x; 1.0055x over previous
import jax
import jax.numpy as jnp
from jax.experimental import pallas as pl
from jax.experimental.pallas import tpu as pltpu

_B = 4
_S = 2048
_D = 768
_CHUNK = 256


def _body(x_ref, q_ref, o_ref, accx, accq):
    i = pl.program_id(0)
    n = pl.num_programs(0)

    @pl.when(i == 0)
    def _init():
        accx[...] = jnp.zeros_like(accx)
        accq[...] = jnp.zeros_like(accq)

    accx[...] += jnp.sum(x_ref[...], axis=1)
    accq[...] += jnp.sum(q_ref[...], axis=1)

    @pl.when(i == n - 1)
    def _finish():
        xp = accx[...] * (1.0 / _S)
        qp = accq[...] * (1.0 / _S)
        attn = jax.lax.dot_general(
            qp, xp, (((1,), (1,)), ((), ())),
            preferred_element_type=jnp.float32)
        attn = jax.nn.softmax(attn, axis=-1)
        ctx = jnp.dot(attn, xp, preferred_element_type=jnp.float32)
        o_ref[...] = ctx[:, None, :]


def kernel(x, query, keys, values, energy_score):
    del keys, values, energy_score
    return pl.pallas_call(
        _body,
        grid=(_S // _CHUNK,),
        in_specs=[
            pl.BlockSpec((_B, _CHUNK, _D), lambda i: (0, i, 0)),
            pl.BlockSpec((_B, _CHUNK, _D), lambda i: (0, i, 0)),
        ],
        out_specs=pl.BlockSpec((_B, 1, _D), lambda i: (0, 0, 0)),
        out_shape=jax.ShapeDtypeStruct((_B, 1, _D), jnp.float32),
        scratch_shapes=[
            pltpu.VMEM((_B, _D), jnp.float32),
            pltpu.VMEM((_B, _D), jnp.float32),
        ],
    )(x, query)
